# Initial kernel scaffold; baseline (speedup 1.0000x reference)
import jax
import jax.numpy as jnp
from jax.experimental import pallas as pl


def kernel(x, edge_index, edge_weight, W1l, b1l, W1r, b1r, att1, We1, bias1, W2l, b2l, W2r, b2r, att2, We2, bias2, Ws, bs, a1, a2):
    raise NotImplementedError("write your pallas kernel here")


# speedup vs baseline: 10.2142x; 10.2142x over previous
import dataclasses
import functools

import jax
import jax.numpy as jnp
from jax import lax
from jax.experimental import pallas as pl
from jax.experimental.pallas import tpu as pltpu
from jax.experimental.pallas import tpu_sc as plsc

_N = 10000
_E = 320000
_D = 128
_NC = 2
_NS = 16
_L = 16
_NW = _NC * _NS
_EPT = _E // _NW
_BB = 80
_NBLK = _EPT // _BB
_NGRP = _BB // _L
_NCH = _D // _L
_NP = 10240
_RPT = _NP // _NS

_RB = 512
_NRB = _NP // _RB


def _make_sc_pass(with_stats):
  mesh = plsc.VectorSubcoreMesh(core_axis_name="c", subcore_axis_name="s")
  n_out = 4 if with_stats else 2
  out_type = [jax.ShapeDtypeStruct((_NC, _NP, _D), jnp.float32),
              jax.ShapeDtypeStruct((_NW, _NP), jnp.float32)]
  if with_stats:
    out_type += [jax.ShapeDtypeStruct((_NW, _NP), jnp.float32),
                 jax.ShapeDtypeStruct((_NW, _NP), jnp.float32)]
  scratch = [
      pltpu.VMEM((_BB, _D), jnp.float32),
      pltpu.VMEM((_BB, _D), jnp.float32),
      pltpu.VMEM((_BB,), jnp.int32),
      pltpu.VMEM((_BB,), jnp.int32),
      pltpu.VMEM((_BB,), jnp.float32),
      pltpu.VMEM((_L, _L), jnp.float32),
      pltpu.VMEM((_NP,), jnp.float32),
      pltpu.VMEM((_D,), jnp.float32),
      pltpu.VMEM((_D,), jnp.float32),
      pltpu.VMEM_SHARED((_NP, _D), jnp.float32),
      pltpu.SemaphoreType.DMA,
      pltpu.SemaphoreType.DMA,
  ]

  def body(xl_hbm, xr_hbm, src_hbm, dst_hbm, ew_hbm, wev_hbm, att_hbm, *rest):
    outs = rest[:n_out]
    num_hbm, den_hbm = outs[0], outs[1]
    if with_stats:
      cnt_hbm, ss_hbm = outs[2], outs[3]
    (xs, xd, sidx, didx, ewb, tmp, denb, wevb, attb, numsh,
     sem0, sem1) = rest[n_out:]
    cid = lax.axis_index("c")
    sid = lax.axis_index("s")
    wid = cid * _NS + sid

    zero16 = jnp.zeros((_L,), jnp.float32)
    ones16 = jnp.ones((_L,), jnp.float32)
    iota16 = lax.iota(jnp.int32, _L)

    def zero_denb():
      @pl.loop(0, _NP // _L)
      def _(i):
        denb[pl.ds(i * _L, _L)] = zero16

    zero_denb()

    @pl.loop(0, _BB)
    def _(r):
      for c in range(_NCH):
        xs[r, pl.ds(c * _L, _L)] = zero16
    rbase = sid * _RPT
    for j in range(_RPT // _BB):
      pltpu.sync_copy(xs, numsh.at[pl.ds(rbase + j * _BB, _BB)])
    plsc.subcore_barrier()

    pltpu.sync_copy(wev_hbm, wevb)
    pltpu.sync_copy(att_hbm, attb)
    WEV = [wevb[pl.ds(c * _L, _L)] for c in range(_NCH)]
    ATT = [attb[pl.ds(c * _L, _L)] for c in range(_NCH)]

    ebase0 = wid * _EPT

    @pl.loop(0, _NBLK)
    def _(blk):
      eb = ebase0 + blk * _BB
      pltpu.sync_copy(src_hbm.at[pl.ds(eb, _BB)], sidx)
      pltpu.sync_copy(dst_hbm.at[pl.ds(eb, _BB)], didx)
      pltpu.sync_copy(ew_hbm.at[pl.ds(eb, _BB)], ewb)
      pltpu.async_copy(xl_hbm.at[sidx], xs, sem0).wait()
      pltpu.async_copy(xr_hbm.at[didx], xd, sem1).wait()

      @pl.loop(0, _NGRP)
      def _(g):
        e0 = g * _L
        ew16 = ewb[pl.ds(e0, _L)]
        ews = [ew16[e] for e in range(_L)]
        accs = [None] * _L
        for c in range(_NCH):
          wv = WEV[c]
          at = ATT[c]
          for e in range(_L):
            v = xs[e0 + e, pl.ds(c * _L, _L)] + xd[e0 + e, pl.ds(c * _L, _L)]
            v = v + ews[e] * wv
            z = jnp.maximum(v, 0.2 * v)
            t = z * at
            accs[e] = t if c == 0 else accs[e] + t
        for e in range(_L):
          tmp[e, pl.ds(0, _L)] = accs[e]
        alpha = None
        for l in range(_L):
          colv = plsc.load_gather(tmp, [iota16, jnp.full((_L,), l, jnp.int32)])
          alpha = colv if l == 0 else alpha + colv
        wvv = jnp.exp(alpha)
        d16 = didx[pl.ds(e0, _L)]
        plsc.addupdate_scatter(denb, [d16], wvv)
        for e in range(_L):
          w = wvv[e]
          for c in range(_NCH):
            xs[e0 + e, pl.ds(c * _L, _L)] = xs[e0 + e, pl.ds(c * _L, _L)] * w
      pltpu.sync_copy(xs, numsh.at[didx], add=True)

    pltpu.sync_copy(denb, den_hbm.at[wid])

    if with_stats:
      zero_denb()

      @pl.loop(0, _NBLK)
      def _(blk):
        eb = ebase0 + blk * _BB
        pltpu.sync_copy(dst_hbm.at[pl.ds(eb, _BB)], didx)

        @pl.loop(0, _NGRP)
        def _(g):
          d16 = didx[pl.ds(g * _L, _L)]
          plsc.addupdate_scatter(denb, [d16], ones16)

      pltpu.sync_copy(denb, cnt_hbm.at[wid])
      zero_denb()

      @pl.loop(0, _NBLK)
      def _(blk):
        eb = ebase0 + blk * _BB
        pltpu.sync_copy(dst_hbm.at[pl.ds(eb, _BB)], didx)
        pltpu.sync_copy(ew_hbm.at[pl.ds(eb, _BB)], ewb)

        @pl.loop(0, _NGRP)
        def _(g):
          d16 = didx[pl.ds(g * _L, _L)]
          plsc.addupdate_scatter(denb, [d16], ewb[pl.ds(g * _L, _L)])

      pltpu.sync_copy(denb, ss_hbm.at[wid])

    plsc.subcore_barrier()
    step = _RPT // 5
    for j in range(5):
      r0 = rbase + j * step
      pltpu.sync_copy(numsh.at[pl.ds(r0, step)],
                      num_hbm.at[cid, pl.ds(r0, step)])

  cp = pltpu.CompilerParams()
  if "needs_layout_passes" in pltpu.CompilerParams.__dataclass_fields__:
    cp = dataclasses.replace(cp, needs_layout_passes=False)
  cp = dataclasses.replace(cp, use_tc_tiling_on_sc=False)
  return pl.kernel(body, out_type=out_type, mesh=mesh, scratch_types=scratch,
                   compiler_params=cp)


_sc_pass_stats = _make_sc_pass(True)
_sc_pass = _make_sc_pass(False)


def _proj3(x, wlT, bl, wrT, br, wsT, bs):
  def body(x_ref, wl_ref, bl_ref, wr_ref, br_ref, ws_ref, bs_ref,
           xl_ref, xr_ref, sk_ref):
    xb = x_ref[...]
    xl_ref[...] = jnp.dot(xb, wl_ref[...],
                          preferred_element_type=jnp.float32) + bl_ref[...]
    xr_ref[...] = jnp.dot(xb, wr_ref[...],
                          preferred_element_type=jnp.float32) + br_ref[...]
    sk_ref[...] = jnp.dot(xb, ws_ref[...],
                          preferred_element_type=jnp.float32) + bs_ref[...]

  row = pl.BlockSpec((_RB, _D), lambda i: (i, 0))
  mat = pl.BlockSpec((_D, _D), lambda i: (0, 0))
  vec = pl.BlockSpec((1, _D), lambda i: (0, 0))
  return pl.pallas_call(
      body,
      grid=(_NRB,),
      in_specs=[row, mat, vec, mat, vec, mat, vec],
      out_specs=[row, row, row],
      out_shape=[jax.ShapeDtypeStruct((_NP, _D), jnp.float32)] * 3,
  )(x, wlT, bl, wrT, br, wsT, bs)


def _self_loop_combine(num_ref, den_ref, la, xl_ref, xr_ref, wev_ref,
                       att_ref, bias_ref):
  num = num_ref[0] + num_ref[1]
  den = jnp.sum(den_ref[...], axis=0)
  xl = xl_ref[...]
  v0 = xl + xr_ref[...] + la[:, None] * wev_ref[...]
  z0 = jnp.maximum(v0, 0.2 * v0)
  a0 = jnp.sum(z0 * att_ref[...], axis=1)
  w0 = jnp.exp(a0)
  num = num + w0[:, None] * xl
  den = den + w0 + 1e-16
  return num / den[:, None] + bias_ref[...]


def _combine1(num1, den1, cnt, ssm, xl1, xr1, wev1, att1, bias1, a1,
              w2lT, b2l, w2rT, b2r):
  def body(num_ref, den_ref, cnt_ref, ss_ref, xl_ref, xr_ref, wev_ref,
           att_ref, bias_ref, a1_ref, w2l_ref, b2l_ref, w2r_ref, b2r_ref,
           xl2_ref, xr2_ref):
    la = (jnp.sum(ss_ref[...], axis=0)
          / jnp.maximum(jnp.sum(cnt_ref[...], axis=0), 1.0))
    h = _self_loop_combine(num_ref, den_ref, la, xl_ref, xr_ref, wev_ref,
                           att_ref, bias_ref)
    h = jnp.where(h >= 0, h, a1_ref[...] * h)
    xl2_ref[...] = jnp.dot(h, w2l_ref[...],
                           preferred_element_type=jnp.float32) + b2l_ref[...]
    xr2_ref[...] = jnp.dot(h, w2r_ref[...],
                           preferred_element_type=jnp.float32) + b2r_ref[...]

  row = pl.BlockSpec((_RB, _D), lambda i: (i, 0))
  mat = pl.BlockSpec((_D, _D), lambda i: (0, 0))
  vec = pl.BlockSpec((1, _D), lambda i: (0, 0))
  par = pl.BlockSpec((_NW, _RB), lambda i: (0, i))
  return pl.pallas_call(
      body,
      grid=(_NRB,),
      in_specs=[pl.BlockSpec((_NC, _RB, _D), lambda i: (0, i, 0)),
                par, par, par, row, row, vec, vec, vec, vec,
                mat, vec, mat, vec],
      out_specs=[row, row],
      out_shape=[jax.ShapeDtypeStruct((_NP, _D), jnp.float32)] * 2,
  )(num1, den1, cnt, ssm, xl1, xr1, wev1, att1, bias1, a1,
    w2lT, b2l, w2rT, b2r)


def _combine2(num2, den2, cnt, ssm, xl2, xr2, wev2, att2, bias2, skip, a2):
  def body(num_ref, den_ref, cnt_ref, ss_ref, xl_ref, xr_ref, wev_ref,
           att_ref, bias_ref, sk_ref, a2_ref, out_ref):
    la = (jnp.sum(ss_ref[...], axis=0)
          / jnp.maximum(jnp.sum(cnt_ref[...], axis=0), 1.0))
    h = _self_loop_combine(num_ref, den_ref, la, xl_ref, xr_ref, wev_ref,
                           att_ref, bias_ref)
    h = h + sk_ref[...]
    out_ref[...] = jnp.where(h >= 0, h, a2_ref[...] * h)

  row = pl.BlockSpec((_RB, _D), lambda i: (i, 0))
  vec = pl.BlockSpec((1, _D), lambda i: (0, 0))
  par = pl.BlockSpec((_NW, _RB), lambda i: (0, i))
  return pl.pallas_call(
      body,
      grid=(_NRB,),
      in_specs=[pl.BlockSpec((_NC, _RB, _D), lambda i: (0, i, 0)),
                par, par, par, row, row, vec, vec, vec, row, vec],
      out_specs=row,
      out_shape=jax.ShapeDtypeStruct((_NP, _D), jnp.float32),
  )(num2, den2, cnt, ssm, xl2, xr2, wev2, att2, bias2, skip, a2)


@jax.jit
def _impl(x, edge_index, edge_weight, W1l, b1l, W1r, b1r, att1, We1, bias1,
          W2l, b2l, W2r, b2r, att2, We2, bias2, Ws, bs, a1, a2):
  src = edge_index[0]
  dst = edge_index[1]
  ew = edge_weight[:, 0]
  wev1 = We1[:, 0]
  attv1 = att1.reshape(-1)
  wev2 = We2[:, 0]
  attv2 = att2.reshape(-1)
  r1 = lambda v: v.reshape(1, _D)

  xp = jnp.pad(x, ((0, _NP - _N), (0, 0)))
  xl1, xr1, skip = _proj3(xp, W1l.T, r1(b1l), W1r.T, r1(b1r), Ws.T, r1(bs))
  num1, den1, cnt, ssm = _sc_pass_stats(xl1, xr1, src, dst, ew, wev1, attv1)
  xl2, xr2 = _combine1(num1, den1, cnt, ssm, xl1, xr1, r1(wev1), r1(attv1),
                       r1(bias1), r1(a1), W2l.T, r1(b2l), W2r.T, r1(b2r))
  num2, den2 = _sc_pass(xl2, xr2, src, dst, ew, wev2, attv2)
  out = _combine2(num2, den2, cnt, ssm, xl2, xr2, r1(wev2), r1(attv2),
                  r1(bias2), skip, r1(a2))
  return out[:_N]


def kernel(x, edge_index, edge_weight, W1l, b1l, W1r, b1r, att1, We1, bias1,
           W2l, b2l, W2r, b2r, att2, We2, bias2, Ws, bs, a1, a2):
  return _impl(x, edge_index, edge_weight, W1l, b1l, W1r, b1r, att1, We1,
               bias1, W2l, b2l, W2r, b2r, att2, We2, bias2, Ws, bs, a1, a2)


# speedup vs baseline: 13.3012x; 1.3022x over previous
import dataclasses
import functools

import jax
import jax.numpy as jnp
from jax import lax
from jax.experimental import pallas as pl
from jax.experimental.pallas import tpu as pltpu
from jax.experimental.pallas import tpu_sc as plsc

_N = 10000
_E = 320000
_D = 128
_NC = 2
_NS = 16
_L = 16
_NW = _NC * _NS
_EPT = _E // _NW
_BB = 80
_NBLK = _EPT // _BB
_NGRP = _BB // _L
_NCH = _D // _L
_NP = 10240
_RPT = _NP // _NS

_RB = 512
_NRB = _NP // _RB


def _make_sc_pass(with_stats):
  mesh = plsc.VectorSubcoreMesh(core_axis_name="c", subcore_axis_name="s")
  n_out = 4 if with_stats else 2
  out_type = [jax.ShapeDtypeStruct((_NC, _NP, _D), jnp.float32),
              jax.ShapeDtypeStruct((_NW, _NP), jnp.float32)]
  if with_stats:
    out_type += [jax.ShapeDtypeStruct((_NW, _NP), jnp.float32),
                 jax.ShapeDtypeStruct((_NW, _NP), jnp.float32)]
  scratch = [
      pltpu.VMEM((_BB, _D), jnp.float32),
      pltpu.VMEM((_BB, _D), jnp.float32),
      pltpu.VMEM((_BB,), jnp.int32),
      pltpu.VMEM((_BB,), jnp.int32),
      pltpu.VMEM((_BB,), jnp.float32),
      pltpu.VMEM((_L, _L), jnp.float32),
      pltpu.VMEM((_NP,), jnp.float32),
      pltpu.VMEM((_D,), jnp.float32),
      pltpu.VMEM((_D,), jnp.float32),
      pltpu.VMEM_SHARED((_NP, _D), jnp.float32),
      pltpu.SemaphoreType.DMA,
      pltpu.SemaphoreType.DMA,
      pltpu.SemaphoreType.DMA,
      pltpu.SemaphoreType.DMA,
      pltpu.SemaphoreType.DMA,
  ]

  def body(xl_hbm, xr_hbm, src_hbm, dst_hbm, ew_hbm, wev_hbm, att_hbm, *rest):
    outs = rest[:n_out]
    num_hbm, den_hbm = outs[0], outs[1]
    if with_stats:
      cnt_hbm, ss_hbm = outs[2], outs[3]
    (xs, xd, sidx, didx, ewb, tmp, denb, wevb, attb, numsh,
     sem0, sem1, sem2, sem3, sem4) = rest[n_out:]
    cid = lax.axis_index("c")
    sid = lax.axis_index("s")
    wid = cid * _NS + sid

    zero16 = jnp.zeros((_L,), jnp.float32)
    ones16 = jnp.ones((_L,), jnp.float32)
    iota16 = lax.iota(jnp.int32, _L)

    def zero_denb():
      @pl.loop(0, _NP // _L)
      def _(i):
        denb[pl.ds(i * _L, _L)] = zero16

    zero_denb()

    @pl.loop(0, _BB)
    def _(r):
      for c in range(_NCH):
        xs[r, pl.ds(c * _L, _L)] = zero16
    rbase = sid * _RPT
    for j in range(_RPT // _BB):
      pltpu.sync_copy(xs, numsh.at[pl.ds(rbase + j * _BB, _BB)])
    plsc.subcore_barrier()

    pltpu.sync_copy(wev_hbm, wevb)
    pltpu.sync_copy(att_hbm, attb)
    WEV = [wevb[pl.ds(c * _L, _L)] for c in range(_NCH)]
    ATT = [attb[pl.ds(c * _L, _L)] for c in range(_NCH)]

    ebase0 = wid * _EPT

    @pl.loop(0, _NBLK)
    def _(blk):
      eb = ebase0 + blk * _BB
      ci = pltpu.async_copy(src_hbm.at[pl.ds(eb, _BB)], sidx, sem2)
      cj = pltpu.async_copy(dst_hbm.at[pl.ds(eb, _BB)], didx, sem3)
      ck = pltpu.async_copy(ew_hbm.at[pl.ds(eb, _BB)], ewb, sem4)
      ci.wait()
      cg = pltpu.async_copy(xl_hbm.at[sidx], xs, sem0)
      cj.wait()
      ch = pltpu.async_copy(xr_hbm.at[didx], xd, sem1)
      ck.wait()
      cg.wait()
      ch.wait()

      @pl.loop(0, _NGRP)
      def _(g):
        e0 = g * _L
        ew16 = ewb[pl.ds(e0, _L)]
        ews = [ew16[e] for e in range(_L)]
        accs = [None] * _L
        for c in range(_NCH):
          wv = WEV[c]
          at = ATT[c]
          for e in range(_L):
            v = xs[e0 + e, pl.ds(c * _L, _L)] + xd[e0 + e, pl.ds(c * _L, _L)]
            v = v + ews[e] * wv
            z = jnp.maximum(v, 0.2 * v)
            t = z * at
            accs[e] = t if c == 0 else accs[e] + t
        for e in range(_L):
          tmp[e, pl.ds(0, _L)] = accs[e]
        alpha = None
        for l in range(_L):
          colv = plsc.load_gather(tmp, [iota16, jnp.full((_L,), l, jnp.int32)])
          alpha = colv if l == 0 else alpha + colv
        wvv = jnp.exp(alpha)
        d16 = didx[pl.ds(e0, _L)]
        plsc.addupdate_scatter(denb, [d16], wvv)
        for e in range(_L):
          w = wvv[e]
          for c in range(_NCH):
            xs[e0 + e, pl.ds(c * _L, _L)] = xs[e0 + e, pl.ds(c * _L, _L)] * w
      pltpu.sync_copy(xs, numsh.at[didx], add=True)

    pltpu.sync_copy(denb, den_hbm.at[wid])

    if with_stats:
      zero_denb()

      @pl.loop(0, _NBLK)
      def _(blk):
        eb = ebase0 + blk * _BB
        pltpu.sync_copy(dst_hbm.at[pl.ds(eb, _BB)], didx)

        @pl.loop(0, _NGRP)
        def _(g):
          d16 = didx[pl.ds(g * _L, _L)]
          plsc.addupdate_scatter(denb, [d16], ones16)

      pltpu.sync_copy(denb, cnt_hbm.at[wid])
      zero_denb()

      @pl.loop(0, _NBLK)
      def _(blk):
        eb = ebase0 + blk * _BB
        pltpu.sync_copy(dst_hbm.at[pl.ds(eb, _BB)], didx)
        pltpu.sync_copy(ew_hbm.at[pl.ds(eb, _BB)], ewb)

        @pl.loop(0, _NGRP)
        def _(g):
          d16 = didx[pl.ds(g * _L, _L)]
          plsc.addupdate_scatter(denb, [d16], ewb[pl.ds(g * _L, _L)])

      pltpu.sync_copy(denb, ss_hbm.at[wid])

    plsc.subcore_barrier()
    step = _RPT // 5
    for j in range(5):
      r0 = rbase + j * step
      pltpu.sync_copy(numsh.at[pl.ds(r0, step)],
                      num_hbm.at[cid, pl.ds(r0, step)])

  cp = pltpu.CompilerParams()
  if "needs_layout_passes" in pltpu.CompilerParams.__dataclass_fields__:
    cp = dataclasses.replace(cp, needs_layout_passes=False)
  cp = dataclasses.replace(cp, use_tc_tiling_on_sc=False)
  return pl.kernel(body, out_type=out_type, mesh=mesh, scratch_types=scratch,
                   compiler_params=cp)


_sc_pass_stats = _make_sc_pass(True)
_sc_pass = _make_sc_pass(False)


def _proj3(x, wlT, bl, wrT, br, wsT, bs):
  def body(x_ref, wl_ref, bl_ref, wr_ref, br_ref, ws_ref, bs_ref,
           xl_ref, xr_ref, sk_ref):
    xb = x_ref[...]
    xl_ref[...] = jnp.dot(xb, wl_ref[...],
                          preferred_element_type=jnp.float32) + bl_ref[...]
    xr_ref[...] = jnp.dot(xb, wr_ref[...],
                          preferred_element_type=jnp.float32) + br_ref[...]
    sk_ref[...] = jnp.dot(xb, ws_ref[...],
                          preferred_element_type=jnp.float32) + bs_ref[...]

  row = pl.BlockSpec((_RB, _D), lambda i: (i, 0))
  mat = pl.BlockSpec((_D, _D), lambda i: (0, 0))
  vec = pl.BlockSpec((1, _D), lambda i: (0, 0))
  return pl.pallas_call(
      body,
      grid=(_NRB,),
      in_specs=[row, mat, vec, mat, vec, mat, vec],
      out_specs=[row, row, row],
      out_shape=[jax.ShapeDtypeStruct((_NP, _D), jnp.float32)] * 3,
  )(x, wlT, bl, wrT, br, wsT, bs)


def _self_loop_combine(num_ref, den_ref, la, xl_ref, xr_ref, wev_ref,
                       att_ref, bias_ref):
  num = num_ref[0] + num_ref[1]
  den = jnp.sum(den_ref[...], axis=0)
  xl = xl_ref[...]
  v0 = xl + xr_ref[...] + la[:, None] * wev_ref[...]
  z0 = jnp.maximum(v0, 0.2 * v0)
  a0 = jnp.sum(z0 * att_ref[...], axis=1)
  w0 = jnp.exp(a0)
  num = num + w0[:, None] * xl
  den = den + w0 + 1e-16
  return num / den[:, None] + bias_ref[...]


def _combine1(num1, den1, cnt, ssm, xl1, xr1, wev1, att1, bias1, a1,
              w2lT, b2l, w2rT, b2r):
  def body(num_ref, den_ref, cnt_ref, ss_ref, xl_ref, xr_ref, wev_ref,
           att_ref, bias_ref, a1_ref, w2l_ref, b2l_ref, w2r_ref, b2r_ref,
           xl2_ref, xr2_ref):
    la = (jnp.sum(ss_ref[...], axis=0)
          / jnp.maximum(jnp.sum(cnt_ref[...], axis=0), 1.0))
    h = _self_loop_combine(num_ref, den_ref, la, xl_ref, xr_ref, wev_ref,
                           att_ref, bias_ref)
    h = jnp.where(h >= 0, h, a1_ref[...] * h)
    xl2_ref[...] = jnp.dot(h, w2l_ref[...],
                           preferred_element_type=jnp.float32) + b2l_ref[...]
    xr2_ref[...] = jnp.dot(h, w2r_ref[...],
                           preferred_element_type=jnp.float32) + b2r_ref[...]

  row = pl.BlockSpec((_RB, _D), lambda i: (i, 0))
  mat = pl.BlockSpec((_D, _D), lambda i: (0, 0))
  vec = pl.BlockSpec((1, _D), lambda i: (0, 0))
  par = pl.BlockSpec((_NW, _RB), lambda i: (0, i))
  return pl.pallas_call(
      body,
      grid=(_NRB,),
      in_specs=[pl.BlockSpec((_NC, _RB, _D), lambda i: (0, i, 0)),
                par, par, par, row, row, vec, vec, vec, vec,
                mat, vec, mat, vec],
      out_specs=[row, row],
      out_shape=[jax.ShapeDtypeStruct((_NP, _D), jnp.float32)] * 2,
  )(num1, den1, cnt, ssm, xl1, xr1, wev1, att1, bias1, a1,
    w2lT, b2l, w2rT, b2r)


def _combine2(num2, den2, cnt, ssm, xl2, xr2, wev2, att2, bias2, skip, a2):
  def body(num_ref, den_ref, cnt_ref, ss_ref, xl_ref, xr_ref, wev_ref,
           att_ref, bias_ref, sk_ref, a2_ref, out_ref):
    la = (jnp.sum(ss_ref[...], axis=0)
          / jnp.maximum(jnp.sum(cnt_ref[...], axis=0), 1.0))
    h = _self_loop_combine(num_ref, den_ref, la, xl_ref, xr_ref, wev_ref,
                           att_ref, bias_ref)
    h = h + sk_ref[...]
    out_ref[...] = jnp.where(h >= 0, h, a2_ref[...] * h)

  row = pl.BlockSpec((_RB, _D), lambda i: (i, 0))
  vec = pl.BlockSpec((1, _D), lambda i: (0, 0))
  par = pl.BlockSpec((_NW, _RB), lambda i: (0, i))
  return pl.pallas_call(
      body,
      grid=(_NRB,),
      in_specs=[pl.BlockSpec((_NC, _RB, _D), lambda i: (0, i, 0)),
                par, par, par, row, row, vec, vec, vec, row, vec],
      out_specs=row,
      out_shape=jax.ShapeDtypeStruct((_NP, _D), jnp.float32),
  )(num2, den2, cnt, ssm, xl2, xr2, wev2, att2, bias2, skip, a2)


@jax.jit
def _impl(x, edge_index, edge_weight, W1l, b1l, W1r, b1r, att1, We1, bias1,
          W2l, b2l, W2r, b2r, att2, We2, bias2, Ws, bs, a1, a2):
  src = edge_index[0]
  dst = edge_index[1]
  ew = edge_weight[:, 0]
  wev1 = We1[:, 0]
  attv1 = att1.reshape(-1)
  wev2 = We2[:, 0]
  attv2 = att2.reshape(-1)
  r1 = lambda v: v.reshape(1, _D)

  xp = jnp.pad(x, ((0, _NP - _N), (0, 0)))
  xl1, xr1, skip = _proj3(xp, W1l.T, r1(b1l), W1r.T, r1(b1r), Ws.T, r1(bs))
  num1, den1, cnt, ssm = _sc_pass_stats(xl1, xr1, src, dst, ew, wev1, attv1)
  xl2, xr2 = _combine1(num1, den1, cnt, ssm, xl1, xr1, r1(wev1), r1(attv1),
                       r1(bias1), r1(a1), W2l.T, r1(b2l), W2r.T, r1(b2r))
  num2, den2 = _sc_pass(xl2, xr2, src, dst, ew, wev2, attv2)
  out = _combine2(num2, den2, cnt, ssm, xl2, xr2, r1(wev2), r1(attv2),
                  r1(bias2), skip, r1(a2))
  return out[:_N]


def kernel(x, edge_index, edge_weight, W1l, b1l, W1r, b1r, att1, We1, bias1,
           W2l, b2l, W2r, b2r, att2, We2, bias2, Ws, bs, a1, a2):
  return _impl(x, edge_index, edge_weight, W1l, b1l, W1r, b1r, att1, We1,
               bias1, W2l, b2l, W2r, b2r, att2, We2, bias2, Ws, bs, a1, a2)


# speedup vs baseline: 18.2248x; 1.3702x over previous
import dataclasses
import functools

import jax
import jax.numpy as jnp
from jax import lax
from jax.experimental import pallas as pl
from jax.experimental.pallas import tpu as pltpu
from jax.experimental.pallas import tpu_sc as plsc

_N = 10000
_E = 320000
_D = 128
_NC = 2
_NS = 16
_L = 16
_NW = _NC * _NS
_EPT = _E // _NW
_BB = 80
_NBLK = _EPT // _BB
_NGRP = _BB // _L
_NCH = _D // _L
_NP = 10240
_RPT = _NP // _NS

_RB = 512
_NRB = _NP // _RB


_SB = 25
_NSB = _NBLK // _SB


def _make_sc_pass(with_stats):
  mesh = plsc.VectorSubcoreMesh(core_axis_name="c", subcore_axis_name="s")
  n_out = 4 if with_stats else 2
  out_type = [jax.ShapeDtypeStruct((_NC, _NP, _D), jnp.float32),
              jax.ShapeDtypeStruct((_NW, _NP), jnp.float32)]
  if with_stats:
    out_type += [jax.ShapeDtypeStruct((_NW, _NP), jnp.float32),
                 jax.ShapeDtypeStruct((_NW, _NP), jnp.float32)]
  scratch = [
      pltpu.VMEM((_BB, _D), jnp.float32),
      pltpu.VMEM((_BB, _D), jnp.float32),
      pltpu.VMEM((_BB, _D), jnp.float32),
      pltpu.VMEM((_SB, _BB), jnp.int32),
      pltpu.VMEM((_SB, _BB), jnp.int32),
      pltpu.VMEM((_SB, _BB), jnp.float32),
      pltpu.VMEM((_L, _L), jnp.float32),
      pltpu.VMEM((_NP,), jnp.float32),
      pltpu.VMEM((_D,), jnp.float32),
      pltpu.VMEM((_D,), jnp.float32),
      pltpu.VMEM_SHARED((_NP, _D), jnp.float32),
  ] + [pltpu.SemaphoreType.DMA] * 8

  def body(xl_hbm, xr_hbm, src_hbm, dst_hbm, ew_hbm, wev_hbm, att_hbm, *rest):
    outs = rest[:n_out]
    num_hbm, den_hbm = outs[0], outs[1]
    if with_stats:
      cnt_hbm, ss_hbm = outs[2], outs[3]
    (xs0, xs1, xd, sidx, didx, ewb, tmp, denb, wevb, attb, numsh,
     semx0, semx1, semd, sems0, sems1, semi0, semi1, semi2) = rest[n_out:]
    cid = lax.axis_index("c")
    sid = lax.axis_index("s")
    wid = cid * _NS + sid

    zero16 = jnp.zeros((_L,), jnp.float32)
    ones16 = jnp.ones((_L,), jnp.float32)
    iota16 = lax.iota(jnp.int32, _L)

    def zero_denb():
      @pl.loop(0, _NP // _L)
      def _(i):
        denb[pl.ds(i * _L, _L)] = zero16

    zero_denb()

    @pl.loop(0, _BB)
    def _(r):
      for c in range(_NCH):
        xs0[r, pl.ds(c * _L, _L)] = zero16
    rbase = sid * _RPT
    for j in range(_RPT // _BB):
      pltpu.sync_copy(xs0, numsh.at[pl.ds(rbase + j * _BB, _BB)])
    plsc.subcore_barrier()

    pltpu.sync_copy(wev_hbm, wevb)
    pltpu.sync_copy(att_hbm, attb)
    WEV = [wevb[pl.ds(c * _L, _L)] for c in range(_NCH)]
    ATT = [attb[pl.ds(c * _L, _L)] for c in range(_NCH)]

    row0 = wid * _NBLK

    def load_idx_super(s, want_src, want_ew):
      r0 = row0 + s * _SB
      cps = []
      if want_src:
        cps.append(pltpu.async_copy(src_hbm.at[pl.ds(r0, _SB)], sidx, semi0))
      cps.append(pltpu.async_copy(dst_hbm.at[pl.ds(r0, _SB)], didx, semi1))
      if want_ew:
        cps.append(pltpu.async_copy(ew_hbm.at[pl.ds(r0, _SB)], ewb, semi2))
      for cp in cps:
        cp.wait()

    def compute_block(j, xc):
      @pl.loop(0, _NGRP)
      def _(g):
        e0 = g * _L
        ew16 = ewb[j, pl.ds(e0, _L)]
        ews = [ew16[e] for e in range(_L)]
        accs = [None] * _L
        for c in range(_NCH):
          wv = WEV[c]
          at = ATT[c]
          for e in range(_L):
            v = xc[e0 + e, pl.ds(c * _L, _L)] + xd[e0 + e, pl.ds(c * _L, _L)]
            v = v + ews[e] * wv
            z = jnp.maximum(v, 0.2 * v)
            t = z * at
            accs[e] = t if c == 0 else accs[e] + t
        for e in range(_L):
          tmp[e, pl.ds(0, _L)] = accs[e]
        alpha = None
        for l in range(_L):
          colv = plsc.load_gather(tmp, [iota16, jnp.full((_L,), l, jnp.int32)])
          alpha = colv if l == 0 else alpha + colv
        wvv = jnp.exp(alpha)
        d16 = didx[j, pl.ds(e0, _L)]
        plsc.addupdate_scatter(denb, [d16], wvv)
        for e in range(_L):
          w = wvv[e]
          for c in range(_NCH):
            xc[e0 + e, pl.ds(c * _L, _L)] = xc[e0 + e, pl.ds(c * _L, _L)] * w

    def start_xs(j, buf, sem):
      return pltpu.async_copy(xl_hbm.at[sidx.at[j]], buf, sem)

    def start_xd(j):
      return pltpu.async_copy(xr_hbm.at[didx.at[j]], xd, semd)

    def wait_xs(buf, sem):
      pltpu.make_async_copy(xl_hbm.at[sidx.at[0]], buf, sem).wait()

    def wait_xd():
      pltpu.make_async_copy(xr_hbm.at[didx.at[0]], xd, semd).wait()

    def start_scat(j, buf, sem):
      return pltpu.async_copy(buf, numsh.at[didx.at[j]], sem, add=True)

    def wait_scat(buf, sem):
      pltpu.make_async_copy(buf, numsh.at[didx.at[0]], sem).wait()

    def half(j, xc, semxc, semsc, xo, semxo, semso):
      wait_xs(xc, semxc)
      wait_xd()
      compute_block(j, xc)
      start_xd(j + 1)
      wait_scat(xo, semso)
      start_xs(j + 1, xo, semxo)
      start_scat(j, xc, semsc)

    @pl.loop(0, _NSB)
    def _(s):
      load_idx_super(s, True, True)
      start_xs(0, xs0, semx0)
      start_xd(0)
      wait_xs(xs0, semx0)
      wait_xd()
      compute_block(0, xs0)
      start_xd(1)
      start_xs(1, xs1, semx1)
      start_scat(0, xs0, sems0)
      @pl.loop(0, (_SB - 3) // 2)
      def _(k):
        j = 2 * k + 1
        half(j, xs1, semx1, sems1, xs0, semx0, sems0)
        half(j + 1, xs0, semx0, sems0, xs1, semx1, sems1)
      half(_SB - 2, xs1, semx1, sems1, xs0, semx0, sems0)
      wait_xs(xs0, semx0)
      wait_xd()
      compute_block(_SB - 1, xs0)
      wait_scat(xs1, sems1)
      pltpu.sync_copy(xs0, numsh.at[didx.at[_SB - 1]], add=True)

    pltpu.sync_copy(denb, den_hbm.at[wid])

    if with_stats:
      zero_denb()

      @pl.loop(0, _NSB)
      def _(s):
        load_idx_super(s, False, False)

        @pl.loop(0, _SB)
        def _(b):
          @pl.loop(0, _NGRP)
          def _(g):
            d16 = didx[b, pl.ds(g * _L, _L)]
            plsc.addupdate_scatter(denb, [d16], ones16)

      pltpu.sync_copy(denb, cnt_hbm.at[wid])
      zero_denb()

      @pl.loop(0, _NSB)
      def _(s):
        load_idx_super(s, False, True)

        @pl.loop(0, _SB)
        def _(b):
          @pl.loop(0, _NGRP)
          def _(g):
            d16 = didx[b, pl.ds(g * _L, _L)]
            plsc.addupdate_scatter(denb, [d16], ewb[b, pl.ds(g * _L, _L)])

      pltpu.sync_copy(denb, ss_hbm.at[wid])

    plsc.subcore_barrier()
    step = _RPT // 5
    for j in range(5):
      r0 = rbase + j * step
      pltpu.sync_copy(numsh.at[pl.ds(r0, step)],
                      num_hbm.at[cid, pl.ds(r0, step)])

  cp = pltpu.CompilerParams()
  if "needs_layout_passes" in pltpu.CompilerParams.__dataclass_fields__:
    cp = dataclasses.replace(cp, needs_layout_passes=False)
  cp = dataclasses.replace(cp, use_tc_tiling_on_sc=False)
  return pl.kernel(body, out_type=out_type, mesh=mesh, scratch_types=scratch,
                   compiler_params=cp)


_sc_pass_stats = _make_sc_pass(True)
_sc_pass = _make_sc_pass(False)


def _proj3(x, wlT, bl, wrT, br, wsT, bs):
  def body(x_ref, wl_ref, bl_ref, wr_ref, br_ref, ws_ref, bs_ref,
           xl_ref, xr_ref, sk_ref):
    xb = x_ref[...]
    xl_ref[...] = jnp.dot(xb, wl_ref[...],
                          preferred_element_type=jnp.float32) + bl_ref[...]
    xr_ref[...] = jnp.dot(xb, wr_ref[...],
                          preferred_element_type=jnp.float32) + br_ref[...]
    sk_ref[...] = jnp.dot(xb, ws_ref[...],
                          preferred_element_type=jnp.float32) + bs_ref[...]

  row = pl.BlockSpec((_RB, _D), lambda i: (i, 0))
  mat = pl.BlockSpec((_D, _D), lambda i: (0, 0))
  vec = pl.BlockSpec((1, _D), lambda i: (0, 0))
  return pl.pallas_call(
      body,
      grid=(_NRB,),
      in_specs=[row, mat, vec, mat, vec, mat, vec],
      out_specs=[row, row, row],
      out_shape=[jax.ShapeDtypeStruct((_NP, _D), jnp.float32)] * 3,
  )(x, wlT, bl, wrT, br, wsT, bs)


def _self_loop_combine(num_ref, den_ref, la, xl_ref, xr_ref, wev_ref,
                       att_ref, bias_ref):
  num = num_ref[0] + num_ref[1]
  den = jnp.sum(den_ref[...], axis=0)
  xl = xl_ref[...]
  v0 = xl + xr_ref[...] + la[:, None] * wev_ref[...]
  z0 = jnp.maximum(v0, 0.2 * v0)
  a0 = jnp.sum(z0 * att_ref[...], axis=1)
  w0 = jnp.exp(a0)
  num = num + w0[:, None] * xl
  den = den + w0 + 1e-16
  return num / den[:, None] + bias_ref[...]


def _combine1(num1, den1, cnt, ssm, xl1, xr1, wev1, att1, bias1, a1,
              w2lT, b2l, w2rT, b2r):
  def body(num_ref, den_ref, cnt_ref, ss_ref, xl_ref, xr_ref, wev_ref,
           att_ref, bias_ref, a1_ref, w2l_ref, b2l_ref, w2r_ref, b2r_ref,
           xl2_ref, xr2_ref):
    la = (jnp.sum(ss_ref[...], axis=0)
          / jnp.maximum(jnp.sum(cnt_ref[...], axis=0), 1.0))
    h = _self_loop_combine(num_ref, den_ref, la, xl_ref, xr_ref, wev_ref,
                           att_ref, bias_ref)
    h = jnp.where(h >= 0, h, a1_ref[...] * h)
    xl2_ref[...] = jnp.dot(h, w2l_ref[...],
                           preferred_element_type=jnp.float32) + b2l_ref[...]
    xr2_ref[...] = jnp.dot(h, w2r_ref[...],
                           preferred_element_type=jnp.float32) + b2r_ref[...]

  row = pl.BlockSpec((_RB, _D), lambda i: (i, 0))
  mat = pl.BlockSpec((_D, _D), lambda i: (0, 0))
  vec = pl.BlockSpec((1, _D), lambda i: (0, 0))
  par = pl.BlockSpec((_NW, _RB), lambda i: (0, i))
  return pl.pallas_call(
      body,
      grid=(_NRB,),
      in_specs=[pl.BlockSpec((_NC, _RB, _D), lambda i: (0, i, 0)),
                par, par, par, row, row, vec, vec, vec, vec,
                mat, vec, mat, vec],
      out_specs=[row, row],
      out_shape=[jax.ShapeDtypeStruct((_NP, _D), jnp.float32)] * 2,
  )(num1, den1, cnt, ssm, xl1, xr1, wev1, att1, bias1, a1,
    w2lT, b2l, w2rT, b2r)


def _combine2(num2, den2, cnt, ssm, xl2, xr2, wev2, att2, bias2, skip, a2):
  def body(num_ref, den_ref, cnt_ref, ss_ref, xl_ref, xr_ref, wev_ref,
           att_ref, bias_ref, sk_ref, a2_ref, out_ref):
    la = (jnp.sum(ss_ref[...], axis=0)
          / jnp.maximum(jnp.sum(cnt_ref[...], axis=0), 1.0))
    h = _self_loop_combine(num_ref, den_ref, la, xl_ref, xr_ref, wev_ref,
                           att_ref, bias_ref)
    h = h + sk_ref[...]
    out_ref[...] = jnp.where(h >= 0, h, a2_ref[...] * h)

  row = pl.BlockSpec((_RB, _D), lambda i: (i, 0))
  vec = pl.BlockSpec((1, _D), lambda i: (0, 0))
  par = pl.BlockSpec((_NW, _RB), lambda i: (0, i))
  return pl.pallas_call(
      body,
      grid=(_NRB,),
      in_specs=[pl.BlockSpec((_NC, _RB, _D), lambda i: (0, i, 0)),
                par, par, par, row, row, vec, vec, vec, row, vec],
      out_specs=row,
      out_shape=jax.ShapeDtypeStruct((_NP, _D), jnp.float32),
  )(num2, den2, cnt, ssm, xl2, xr2, wev2, att2, bias2, skip, a2)


@jax.jit
def _impl(x, edge_index, edge_weight, W1l, b1l, W1r, b1r, att1, We1, bias1,
          W2l, b2l, W2r, b2r, att2, We2, bias2, Ws, bs, a1, a2):
  src = edge_index[0].reshape(_E // _BB, _BB)
  dst = edge_index[1].reshape(_E // _BB, _BB)
  ew = edge_weight[:, 0].reshape(_E // _BB, _BB)
  wev1 = We1[:, 0]
  attv1 = att1.reshape(-1)
  wev2 = We2[:, 0]
  attv2 = att2.reshape(-1)
  r1 = lambda v: v.reshape(1, _D)

  xp = jnp.pad(x, ((0, _NP - _N), (0, 0)))
  xl1, xr1, skip = _proj3(xp, W1l.T, r1(b1l), W1r.T, r1(b1r), Ws.T, r1(bs))
  num1, den1, cnt, ssm = _sc_pass_stats(xl1, xr1, src, dst, ew, wev1, attv1)
  xl2, xr2 = _combine1(num1, den1, cnt, ssm, xl1, xr1, r1(wev1), r1(attv1),
                       r1(bias1), r1(a1), W2l.T, r1(b2l), W2r.T, r1(b2r))
  num2, den2 = _sc_pass(xl2, xr2, src, dst, ew, wev2, attv2)
  out = _combine2(num2, den2, cnt, ssm, xl2, xr2, r1(wev2), r1(attv2),
                  r1(bias2), skip, r1(a2))
  return out[:_N]


def kernel(x, edge_index, edge_weight, W1l, b1l, W1r, b1r, att1, We1, bias1,
           W2l, b2l, W2r, b2r, att2, We2, bias2, Ws, bs, a1, a2):
  return _impl(x, edge_index, edge_weight, W1l, b1l, W1r, b1r, att1, We1,
               bias1, W2l, b2l, W2r, b2r, att2, We2, bias2, Ws, bs, a1, a2)


# speedup vs baseline: 19.9756x; 1.0961x over previous
import dataclasses
import functools

import jax
import jax.numpy as jnp
from jax import lax
from jax.experimental import pallas as pl
from jax.experimental.pallas import tpu as pltpu
from jax.experimental.pallas import tpu_sc as plsc

_N = 10000
_E = 320000
_D = 128
_NC = 2
_NS = 16
_L = 16
_NW = _NC * _NS
_EPT = _E // _NW
_BB = 80
_NBLK = _EPT // _BB
_NGRP = _BB // _L
_NCH = _D // _L
_NP = 10240
_RPT = _NP // _NS

_RB = 512
_NRB = _NP // _RB


_SB = 25
_NSB = _NBLK // _SB


def _make_sc_pass(with_stats):
  mesh = plsc.VectorSubcoreMesh(core_axis_name="c", subcore_axis_name="s")
  n_out = 4 if with_stats else 2
  out_type = [jax.ShapeDtypeStruct((_NC, _NP, _D), jnp.float32),
              jax.ShapeDtypeStruct((_NW, _NP), jnp.float32)]
  if with_stats:
    out_type += [jax.ShapeDtypeStruct((_NW, _NP), jnp.float32),
                 jax.ShapeDtypeStruct((_NW, _NP), jnp.float32)]
  scratch = [
      pltpu.VMEM((_BB, _D), jnp.float32),
      pltpu.VMEM((_BB, _D), jnp.float32),
      pltpu.VMEM((_BB, _D), jnp.float32),
      pltpu.VMEM((_SB, _BB), jnp.int32),
      pltpu.VMEM((_SB, _BB), jnp.int32),
      pltpu.VMEM((_SB, _BB), jnp.float32),
      pltpu.VMEM((_L, _L), jnp.float32),
      pltpu.VMEM((_BB,), jnp.float32),
      pltpu.VMEM((_NP,), jnp.float32),
      pltpu.VMEM((_D,), jnp.float32),
      pltpu.VMEM((_D,), jnp.float32),
      pltpu.VMEM_SHARED((_NP, _D), jnp.float32),
  ] + [pltpu.SemaphoreType.DMA] * 8

  def body(xl_hbm, xr_hbm, src_hbm, dst_hbm, ew_hbm, wev_hbm, att_hbm, *rest):
    outs = rest[:n_out]
    num_hbm, den_hbm = outs[0], outs[1]
    if with_stats:
      cnt_hbm, ss_hbm = outs[2], outs[3]
    (xs0, xs1, xd, sidx, didx, ewb, tmp, wb, denb, wevb, attb, numsh,
     semx0, semx1, semd, sems0, sems1, semi0, semi1, semi2) = rest[n_out:]
    cid = lax.axis_index("c")
    sid = lax.axis_index("s")
    wid = cid * _NS + sid

    zero16 = jnp.zeros((_L,), jnp.float32)
    ones16 = jnp.ones((_L,), jnp.float32)
    iota16 = lax.iota(jnp.int32, _L)

    def zero_denb():
      @pl.loop(0, _NP // _L)
      def _(i):
        denb[pl.ds(i * _L, _L)] = zero16

    zero_denb()

    @pl.loop(0, _BB)
    def _(r):
      for c in range(_NCH):
        xs0[r, pl.ds(c * _L, _L)] = zero16
    rbase = sid * _RPT
    for j in range(_RPT // _BB):
      pltpu.sync_copy(xs0, numsh.at[pl.ds(rbase + j * _BB, _BB)])
    plsc.subcore_barrier()

    pltpu.sync_copy(wev_hbm, wevb)
    pltpu.sync_copy(att_hbm, attb)
    WEV = [wevb[pl.ds(c * _L, _L)] for c in range(_NCH)]
    ATT = [attb[pl.ds(c * _L, _L)] for c in range(_NCH)]

    row0 = wid * _NBLK

    def load_idx_super(s, want_src, want_ew):
      r0 = row0 + s * _SB
      cps = []
      if want_src:
        cps.append(pltpu.async_copy(src_hbm.at[pl.ds(r0, _SB)], sidx, semi0))
      cps.append(pltpu.async_copy(dst_hbm.at[pl.ds(r0, _SB)], didx, semi1))
      if want_ew:
        cps.append(pltpu.async_copy(ew_hbm.at[pl.ds(r0, _SB)], ewb, semi2))
      for cp in cps:
        cp.wait()

    def compute_block(j, xc):
      @pl.loop(0, _NGRP)
      def _(g):
        e0 = g * _L
        ew16 = ewb[j, pl.ds(e0, _L)]
        ews = [ew16[e] for e in range(_L)]
        accs = [None] * _L
        for c in range(_NCH):
          wv = WEV[c]
          at = ATT[c]
          for e in range(_L):
            v = xc[e0 + e, pl.ds(c * _L, _L)] + xd[e0 + e, pl.ds(c * _L, _L)]
            v = v + ews[e] * wv
            z = jnp.maximum(v, 0.2 * v)
            t = z * at
            accs[e] = t if c == 0 else accs[e] + t
        for e in range(_L):
          tmp[e, pl.ds(0, _L)] = accs[e]
        alpha = None
        for l in range(_L):
          colv = plsc.load_gather(tmp, [iota16, jnp.full((_L,), l, jnp.int32)])
          alpha = colv if l == 0 else alpha + colv
        wvv = jnp.exp(alpha)
        d16 = didx[j, pl.ds(e0, _L)]
        plsc.addupdate_scatter(denb, [d16], wvv)
        wb[pl.ds(e0, _L)] = wvv

    def scale_block(j, xc):
      @pl.loop(0, _NGRP)
      def _(g):
        e0 = g * _L
        wvv = wb[pl.ds(e0, _L)]
        for e in range(_L):
          w = wvv[e]
          for c in range(_NCH):
            xc[e0 + e, pl.ds(c * _L, _L)] = xc[e0 + e, pl.ds(c * _L, _L)] * w

    def start_xs(j, buf, sem):
      return pltpu.async_copy(xl_hbm.at[sidx.at[j]], buf, sem)

    def start_xd(j):
      return pltpu.async_copy(xr_hbm.at[didx.at[j]], xd, semd)

    def wait_xs(buf, sem):
      pltpu.make_async_copy(xl_hbm.at[sidx.at[0]], buf, sem).wait()

    def wait_xd():
      pltpu.make_async_copy(xr_hbm.at[didx.at[0]], xd, semd).wait()

    def start_scat(j, buf, sem):
      return pltpu.async_copy(buf, numsh.at[didx.at[j]], sem, add=True)

    def wait_scat(buf, sem):
      pltpu.make_async_copy(buf, numsh.at[didx.at[0]], sem).wait()

    def half(j, xc, semxc, semsc, xo, semxo, semso):
      wait_xs(xc, semxc)
      wait_xd()
      compute_block(j, xc)
      start_xd(j + 1)
      wait_scat(xo, semso)
      start_xs(j + 1, xo, semxo)
      scale_block(j, xc)
      start_scat(j, xc, semsc)

    @pl.loop(0, _NSB)
    def _(s):
      load_idx_super(s, True, True)
      start_xs(0, xs0, semx0)
      start_xd(0)
      wait_xs(xs0, semx0)
      wait_xd()
      compute_block(0, xs0)
      start_xd(1)
      start_xs(1, xs1, semx1)
      scale_block(0, xs0)
      start_scat(0, xs0, sems0)
      @pl.loop(0, (_SB - 3) // 2)
      def _(k):
        j = 2 * k + 1
        half(j, xs1, semx1, sems1, xs0, semx0, sems0)
        half(j + 1, xs0, semx0, sems0, xs1, semx1, sems1)
      half(_SB - 2, xs1, semx1, sems1, xs0, semx0, sems0)
      wait_xs(xs0, semx0)
      wait_xd()
      compute_block(_SB - 1, xs0)
      wait_scat(xs1, sems1)
      scale_block(_SB - 1, xs0)
      pltpu.sync_copy(xs0, numsh.at[didx.at[_SB - 1]], add=True)

    pltpu.sync_copy(denb, den_hbm.at[wid])

    if with_stats:
      zero_denb()

      @pl.loop(0, _NSB)
      def _(s):
        load_idx_super(s, False, False)

        @pl.loop(0, _SB)
        def _(b):
          @pl.loop(0, _NGRP)
          def _(g):
            d16 = didx[b, pl.ds(g * _L, _L)]
            plsc.addupdate_scatter(denb, [d16], ones16)

      pltpu.sync_copy(denb, cnt_hbm.at[wid])
      zero_denb()

      @pl.loop(0, _NSB)
      def _(s):
        load_idx_super(s, False, True)

        @pl.loop(0, _SB)
        def _(b):
          @pl.loop(0, _NGRP)
          def _(g):
            d16 = didx[b, pl.ds(g * _L, _L)]
            plsc.addupdate_scatter(denb, [d16], ewb[b, pl.ds(g * _L, _L)])

      pltpu.sync_copy(denb, ss_hbm.at[wid])

    plsc.subcore_barrier()
    step = _RPT // 5
    for j in range(5):
      r0 = rbase + j * step
      pltpu.sync_copy(numsh.at[pl.ds(r0, step)],
                      num_hbm.at[cid, pl.ds(r0, step)])

  cp = pltpu.CompilerParams()
  if "needs_layout_passes" in pltpu.CompilerParams.__dataclass_fields__:
    cp = dataclasses.replace(cp, needs_layout_passes=False)
  cp = dataclasses.replace(cp, use_tc_tiling_on_sc=False)
  return pl.kernel(body, out_type=out_type, mesh=mesh, scratch_types=scratch,
                   compiler_params=cp)


_sc_pass_stats = _make_sc_pass(True)
_sc_pass = _make_sc_pass(False)


def _proj3(x, wlT, bl, wrT, br, wsT, bs):
  def body(x_ref, wl_ref, bl_ref, wr_ref, br_ref, ws_ref, bs_ref,
           xl_ref, xr_ref, sk_ref):
    xb = x_ref[...]
    xl_ref[...] = jnp.dot(xb, wl_ref[...],
                          preferred_element_type=jnp.float32) + bl_ref[...]
    xr_ref[...] = jnp.dot(xb, wr_ref[...],
                          preferred_element_type=jnp.float32) + br_ref[...]
    sk_ref[...] = jnp.dot(xb, ws_ref[...],
                          preferred_element_type=jnp.float32) + bs_ref[...]

  row = pl.BlockSpec((_RB, _D), lambda i: (i, 0))
  mat = pl.BlockSpec((_D, _D), lambda i: (0, 0))
  vec = pl.BlockSpec((1, _D), lambda i: (0, 0))
  return pl.pallas_call(
      body,
      grid=(_NRB,),
      in_specs=[row, mat, vec, mat, vec, mat, vec],
      out_specs=[row, row, row],
      out_shape=[jax.ShapeDtypeStruct((_NP, _D), jnp.float32)] * 3,
  )(x, wlT, bl, wrT, br, wsT, bs)


def _self_loop_combine(num_ref, den_ref, la, xl_ref, xr_ref, wev_ref,
                       att_ref, bias_ref):
  num = num_ref[0] + num_ref[1]
  den = jnp.sum(den_ref[...], axis=0)
  xl = xl_ref[...]
  v0 = xl + xr_ref[...] + la[:, None] * wev_ref[...]
  z0 = jnp.maximum(v0, 0.2 * v0)
  a0 = jnp.sum(z0 * att_ref[...], axis=1)
  w0 = jnp.exp(a0)
  num = num + w0[:, None] * xl
  den = den + w0 + 1e-16
  return num / den[:, None] + bias_ref[...]


def _combine1(num1, den1, cnt, ssm, xl1, xr1, wev1, att1, bias1, a1,
              w2lT, b2l, w2rT, b2r):
  def body(num_ref, den_ref, cnt_ref, ss_ref, xl_ref, xr_ref, wev_ref,
           att_ref, bias_ref, a1_ref, w2l_ref, b2l_ref, w2r_ref, b2r_ref,
           xl2_ref, xr2_ref):
    la = (jnp.sum(ss_ref[...], axis=0)
          / jnp.maximum(jnp.sum(cnt_ref[...], axis=0), 1.0))
    h = _self_loop_combine(num_ref, den_ref, la, xl_ref, xr_ref, wev_ref,
                           att_ref, bias_ref)
    h = jnp.where(h >= 0, h, a1_ref[...] * h)
    xl2_ref[...] = jnp.dot(h, w2l_ref[...],
                           preferred_element_type=jnp.float32) + b2l_ref[...]
    xr2_ref[...] = jnp.dot(h, w2r_ref[...],
                           preferred_element_type=jnp.float32) + b2r_ref[...]

  row = pl.BlockSpec((_RB, _D), lambda i: (i, 0))
  mat = pl.BlockSpec((_D, _D), lambda i: (0, 0))
  vec = pl.BlockSpec((1, _D), lambda i: (0, 0))
  par = pl.BlockSpec((_NW, _RB), lambda i: (0, i))
  return pl.pallas_call(
      body,
      grid=(_NRB,),
      in_specs=[pl.BlockSpec((_NC, _RB, _D), lambda i: (0, i, 0)),
                par, par, par, row, row, vec, vec, vec, vec,
                mat, vec, mat, vec],
      out_specs=[row, row],
      out_shape=[jax.ShapeDtypeStruct((_NP, _D), jnp.float32)] * 2,
  )(num1, den1, cnt, ssm, xl1, xr1, wev1, att1, bias1, a1,
    w2lT, b2l, w2rT, b2r)


def _combine2(num2, den2, cnt, ssm, xl2, xr2, wev2, att2, bias2, skip, a2):
  def body(num_ref, den_ref, cnt_ref, ss_ref, xl_ref, xr_ref, wev_ref,
           att_ref, bias_ref, sk_ref, a2_ref, out_ref):
    la = (jnp.sum(ss_ref[...], axis=0)
          / jnp.maximum(jnp.sum(cnt_ref[...], axis=0), 1.0))
    h = _self_loop_combine(num_ref, den_ref, la, xl_ref, xr_ref, wev_ref,
                           att_ref, bias_ref)
    h = h + sk_ref[...]
    out_ref[...] = jnp.where(h >= 0, h, a2_ref[...] * h)

  row = pl.BlockSpec((_RB, _D), lambda i: (i, 0))
  vec = pl.BlockSpec((1, _D), lambda i: (0, 0))
  par = pl.BlockSpec((_NW, _RB), lambda i: (0, i))
  return pl.pallas_call(
      body,
      grid=(_NRB,),
      in_specs=[pl.BlockSpec((_NC, _RB, _D), lambda i: (0, i, 0)),
                par, par, par, row, row, vec, vec, vec, row, vec],
      out_specs=row,
      out_shape=jax.ShapeDtypeStruct((_NP, _D), jnp.float32),
  )(num2, den2, cnt, ssm, xl2, xr2, wev2, att2, bias2, skip, a2)


@jax.jit
def _impl(x, edge_index, edge_weight, W1l, b1l, W1r, b1r, att1, We1, bias1,
          W2l, b2l, W2r, b2r, att2, We2, bias2, Ws, bs, a1, a2):
  src = edge_index[0].reshape(_E // _BB, _BB)
  dst = edge_index[1].reshape(_E // _BB, _BB)
  ew = edge_weight[:, 0].reshape(_E // _BB, _BB)
  wev1 = We1[:, 0]
  attv1 = att1.reshape(-1)
  wev2 = We2[:, 0]
  attv2 = att2.reshape(-1)
  r1 = lambda v: v.reshape(1, _D)

  xp = jnp.pad(x, ((0, _NP - _N), (0, 0)))
  xl1, xr1, skip = _proj3(xp, W1l.T, r1(b1l), W1r.T, r1(b1r), Ws.T, r1(bs))
  num1, den1, cnt, ssm = _sc_pass_stats(xl1, xr1, src, dst, ew, wev1, attv1)
  xl2, xr2 = _combine1(num1, den1, cnt, ssm, xl1, xr1, r1(wev1), r1(attv1),
                       r1(bias1), r1(a1), W2l.T, r1(b2l), W2r.T, r1(b2r))
  num2, den2 = _sc_pass(xl2, xr2, src, dst, ew, wev2, attv2)
  out = _combine2(num2, den2, cnt, ssm, xl2, xr2, r1(wev2), r1(attv2),
                  r1(bias2), skip, r1(a2))
  return out[:_N]


def kernel(x, edge_index, edge_weight, W1l, b1l, W1r, b1r, att1, We1, bias1,
           W2l, b2l, W2r, b2r, att2, We2, bias2, Ws, bs, a1, a2):
  return _impl(x, edge_index, edge_weight, W1l, b1l, W1r, b1r, att1, We1,
               bias1, W2l, b2l, W2r, b2r, att2, We2, bias2, Ws, bs, a1, a2)


# speedup vs baseline: 19.9773x; 1.0001x over previous
import dataclasses

import jax
import jax.numpy as jnp
from jax import lax
from jax.experimental import pallas as pl
from jax.experimental.pallas import tpu as pltpu
from jax.experimental.pallas import tpu_sc as plsc

_N = 10000
_E = 320000
_D = 128
_NC = 2
_NS = 16
_L = 16
_NW = _NC * _NS
_EPT = _E // _NW
_BB = 80
_NBLK = _EPT // _BB
_NGRP = _BB // _L
_NCH = _D // _L
_NP = 10240
_RPT = _NP // _NS

_RB = 512
_NRB = _NP // _RB


_SB = 25
_NSB = _NBLK // _SB


def _make_sc_pass(with_stats):
  mesh = plsc.VectorSubcoreMesh(core_axis_name="c", subcore_axis_name="s")
  n_out = 4 if with_stats else 2
  out_type = [jax.ShapeDtypeStruct((_NC, _NP, _D), jnp.float32),
              jax.ShapeDtypeStruct((_NW, _NP), jnp.float32)]
  if with_stats:
    out_type += [jax.ShapeDtypeStruct((_NW, _NP), jnp.float32),
                 jax.ShapeDtypeStruct((_NW, _NP), jnp.float32)]
  scratch = [
      pltpu.VMEM((_BB, _D), jnp.float32),
      pltpu.VMEM((_BB, _D), jnp.float32),
      pltpu.VMEM((_BB, _D), jnp.float32),
      pltpu.VMEM((_SB, _BB), jnp.int32),
      pltpu.VMEM((_SB, _BB), jnp.int32),
      pltpu.VMEM((_SB, _BB), jnp.float32),
      pltpu.VMEM((_L, _L), jnp.float32),
      pltpu.VMEM((_BB,), jnp.float32),
      pltpu.VMEM((_NP,), jnp.float32),
      pltpu.VMEM((_D,), jnp.float32),
      pltpu.VMEM((_D,), jnp.float32),
      pltpu.VMEM_SHARED((_NP, _D), jnp.float32),
  ] + [pltpu.SemaphoreType.DMA] * 8

  def body(xl_hbm, xr_hbm, src_hbm, dst_hbm, ew_hbm, wev_hbm, att_hbm, *rest):
    outs = rest[:n_out]
    num_hbm, den_hbm = outs[0], outs[1]
    if with_stats:
      cnt_hbm, ss_hbm = outs[2], outs[3]
    (xs0, xs1, xd, sidx, didx, ewb, tmp, wb, denb, wevb, attb, numsh,
     semx0, semx1, semd, sems0, sems1, semi0, semi1, semi2) = rest[n_out:]
    cid = lax.axis_index("c")
    sid = lax.axis_index("s")
    wid = cid * _NS + sid

    zero16 = jnp.zeros((_L,), jnp.float32)
    ones16 = jnp.ones((_L,), jnp.float32)
    iota16 = lax.iota(jnp.int32, _L)

    def zero_denb():
      @pl.loop(0, _NP // _L)
      def _(i):
        denb[pl.ds(i * _L, _L)] = zero16

    zero_denb()

    @pl.loop(0, _BB)
    def _(r):
      for c in range(_NCH):
        xs0[r, pl.ds(c * _L, _L)] = zero16
    rbase = sid * _RPT
    for j in range(_RPT // _BB):
      pltpu.sync_copy(xs0, numsh.at[pl.ds(rbase + j * _BB, _BB)])
    plsc.subcore_barrier()

    pltpu.sync_copy(wev_hbm, wevb)
    pltpu.sync_copy(att_hbm, attb)
    WEV = [wevb[pl.ds(c * _L, _L)] for c in range(_NCH)]
    ATT = [attb[pl.ds(c * _L, _L)] for c in range(_NCH)]

    row0 = wid * _NBLK

    def load_idx_super(s, want_src, want_ew):
      r0 = row0 + s * _SB
      cps = []
      if want_src:
        cps.append(pltpu.async_copy(src_hbm.at[pl.ds(r0, _SB)], sidx, semi0))
      cps.append(pltpu.async_copy(dst_hbm.at[pl.ds(r0, _SB)], didx, semi1))
      if want_ew:
        cps.append(pltpu.async_copy(ew_hbm.at[pl.ds(r0, _SB)], ewb, semi2))
      for cp in cps:
        cp.wait()

    def compute_block(j, xc):
      @pl.loop(0, _NGRP)
      def _(g):
        e0 = g * _L
        ew16 = ewb[j, pl.ds(e0, _L)]
        ews = [ew16[e] for e in range(_L)]
        accs = [None] * _L
        for c in range(_NCH):
          wv = WEV[c]
          at = ATT[c]
          for e in range(_L):
            v = xc[e0 + e, pl.ds(c * _L, _L)] + xd[e0 + e, pl.ds(c * _L, _L)]
            v = v + ews[e] * wv
            z = jnp.maximum(v, 0.2 * v)
            t = z * at
            accs[e] = t if c == 0 else accs[e] + t
        for e in range(_L):
          tmp[e, pl.ds(0, _L)] = accs[e]
        alpha = None
        for l in range(_L):
          colv = plsc.load_gather(tmp, [iota16, jnp.full((_L,), l, jnp.int32)])
          alpha = colv if l == 0 else alpha + colv
        wvv = jnp.exp(alpha)
        d16 = didx[j, pl.ds(e0, _L)]
        plsc.addupdate_scatter(denb, [d16], wvv)
        wb[pl.ds(e0, _L)] = wvv

    def scale_block(j, xc):
      @pl.loop(0, _NGRP)
      def _(g):
        e0 = g * _L
        wvv = wb[pl.ds(e0, _L)]
        for e in range(_L):
          w = wvv[e]
          for c in range(_NCH):
            xc[e0 + e, pl.ds(c * _L, _L)] = xc[e0 + e, pl.ds(c * _L, _L)] * w

    def start_xs(j, buf, sem):
      return pltpu.async_copy(xl_hbm.at[sidx.at[j]], buf, sem)

    def start_xd(j):
      return pltpu.async_copy(xr_hbm.at[didx.at[j]], xd, semd)

    def wait_xs(buf, sem):
      pltpu.make_async_copy(xl_hbm.at[sidx.at[0]], buf, sem).wait()

    def wait_xd():
      pltpu.make_async_copy(xr_hbm.at[didx.at[0]], xd, semd).wait()

    def start_scat(j, buf, sem):
      return pltpu.async_copy(buf, numsh.at[didx.at[j]], sem, add=True)

    def wait_scat(buf, sem):
      pltpu.make_async_copy(buf, numsh.at[didx.at[0]], sem).wait()

    def half(j, xc, semxc, semsc, xo, semxo, semso):
      wait_xs(xc, semxc)
      wait_xd()
      compute_block(j, xc)
      start_xd(j + 1)
      wait_scat(xo, semso)
      start_xs(j + 1, xo, semxo)
      scale_block(j, xc)
      start_scat(j, xc, semsc)

    @pl.loop(0, _NSB)
    def _(s):
      load_idx_super(s, True, True)
      start_xs(0, xs0, semx0)
      start_xd(0)
      wait_xs(xs0, semx0)
      wait_xd()
      compute_block(0, xs0)
      start_xd(1)
      start_xs(1, xs1, semx1)
      scale_block(0, xs0)
      start_scat(0, xs0, sems0)
      @pl.loop(0, (_SB - 3) // 2)
      def _(k):
        j = 2 * k + 1
        half(j, xs1, semx1, sems1, xs0, semx0, sems0)
        half(j + 1, xs0, semx0, sems0, xs1, semx1, sems1)
      half(_SB - 2, xs1, semx1, sems1, xs0, semx0, sems0)
      wait_xs(xs0, semx0)
      wait_xd()
      compute_block(_SB - 1, xs0)
      wait_scat(xs1, sems1)
      scale_block(_SB - 1, xs0)
      pltpu.sync_copy(xs0, numsh.at[didx.at[_SB - 1]], add=True)

    pltpu.sync_copy(denb, den_hbm.at[wid])

    if with_stats:
      zero_denb()

      @pl.loop(0, _NSB)
      def _(s):
        load_idx_super(s, False, False)

        @pl.loop(0, _SB)
        def _(b):
          @pl.loop(0, _NGRP)
          def _(g):
            d16 = didx[b, pl.ds(g * _L, _L)]
            plsc.addupdate_scatter(denb, [d16], ones16)

      pltpu.sync_copy(denb, cnt_hbm.at[wid])
      zero_denb()

      @pl.loop(0, _NSB)
      def _(s):
        load_idx_super(s, False, True)

        @pl.loop(0, _SB)
        def _(b):
          @pl.loop(0, _NGRP)
          def _(g):
            d16 = didx[b, pl.ds(g * _L, _L)]
            plsc.addupdate_scatter(denb, [d16], ewb[b, pl.ds(g * _L, _L)])

      pltpu.sync_copy(denb, ss_hbm.at[wid])

    plsc.subcore_barrier()
    step = _RPT // 5
    for j in range(5):
      r0 = rbase + j * step
      pltpu.sync_copy(numsh.at[pl.ds(r0, step)],
                      num_hbm.at[cid, pl.ds(r0, step)])

  cp = pltpu.CompilerParams()
  if "needs_layout_passes" in pltpu.CompilerParams.__dataclass_fields__:
    cp = dataclasses.replace(cp, needs_layout_passes=False)
  cp = dataclasses.replace(cp, use_tc_tiling_on_sc=False)
  return pl.kernel(body, out_type=out_type, mesh=mesh, scratch_types=scratch,
                   compiler_params=cp)


_sc_pass_stats = _make_sc_pass(True)
_sc_pass = _make_sc_pass(False)


def _proj3(x, wlT, bl, wrT, br, wsT, bs):
  def body(x_ref, wl_ref, bl_ref, wr_ref, br_ref, ws_ref, bs_ref,
           xl_ref, xr_ref, sk_ref):
    xb = x_ref[...]
    xl_ref[...] = jnp.dot(xb, wl_ref[...],
                          preferred_element_type=jnp.float32) + bl_ref[...]
    xr_ref[...] = jnp.dot(xb, wr_ref[...],
                          preferred_element_type=jnp.float32) + br_ref[...]
    sk_ref[...] = jnp.dot(xb, ws_ref[...],
                          preferred_element_type=jnp.float32) + bs_ref[...]

  row = pl.BlockSpec((_RB, _D), lambda i: (i, 0))
  mat = pl.BlockSpec((_D, _D), lambda i: (0, 0))
  vec = pl.BlockSpec((1, _D), lambda i: (0, 0))
  return pl.pallas_call(
      body,
      grid=(_NRB,),
      in_specs=[row, mat, vec, mat, vec, mat, vec],
      out_specs=[row, row, row],
      out_shape=[jax.ShapeDtypeStruct((_NP, _D), jnp.float32)] * 3,
  )(x, wlT, bl, wrT, br, wsT, bs)


def _self_loop_combine(num_ref, den_ref, la, xl_ref, xr_ref, wev_ref,
                       att_ref, bias_ref):
  num = num_ref[0] + num_ref[1]
  den = jnp.sum(den_ref[...], axis=0)
  xl = xl_ref[...]
  v0 = xl + xr_ref[...] + la[:, None] * wev_ref[...]
  z0 = jnp.maximum(v0, 0.2 * v0)
  a0 = jnp.sum(z0 * att_ref[...], axis=1)
  w0 = jnp.exp(a0)
  num = num + w0[:, None] * xl
  den = den + w0 + 1e-16
  return num / den[:, None] + bias_ref[...]


def _combine1(num1, den1, cnt, ssm, xl1, xr1, wev1, att1, bias1, a1,
              w2lT, b2l, w2rT, b2r):
  def body(num_ref, den_ref, cnt_ref, ss_ref, xl_ref, xr_ref, wev_ref,
           att_ref, bias_ref, a1_ref, w2l_ref, b2l_ref, w2r_ref, b2r_ref,
           xl2_ref, xr2_ref):
    la = (jnp.sum(ss_ref[...], axis=0)
          / jnp.maximum(jnp.sum(cnt_ref[...], axis=0), 1.0))
    h = _self_loop_combine(num_ref, den_ref, la, xl_ref, xr_ref, wev_ref,
                           att_ref, bias_ref)
    h = jnp.where(h >= 0, h, a1_ref[...] * h)
    xl2_ref[...] = jnp.dot(h, w2l_ref[...],
                           preferred_element_type=jnp.float32) + b2l_ref[...]
    xr2_ref[...] = jnp.dot(h, w2r_ref[...],
                           preferred_element_type=jnp.float32) + b2r_ref[...]

  row = pl.BlockSpec((_RB, _D), lambda i: (i, 0))
  mat = pl.BlockSpec((_D, _D), lambda i: (0, 0))
  vec = pl.BlockSpec((1, _D), lambda i: (0, 0))
  par = pl.BlockSpec((_NW, _RB), lambda i: (0, i))
  return pl.pallas_call(
      body,
      grid=(_NRB,),
      in_specs=[pl.BlockSpec((_NC, _RB, _D), lambda i: (0, i, 0)),
                par, par, par, row, row, vec, vec, vec, vec,
                mat, vec, mat, vec],
      out_specs=[row, row],
      out_shape=[jax.ShapeDtypeStruct((_NP, _D), jnp.float32)] * 2,
  )(num1, den1, cnt, ssm, xl1, xr1, wev1, att1, bias1, a1,
    w2lT, b2l, w2rT, b2r)


def _combine2(num2, den2, cnt, ssm, xl2, xr2, wev2, att2, bias2, skip, a2):
  def body(num_ref, den_ref, cnt_ref, ss_ref, xl_ref, xr_ref, wev_ref,
           att_ref, bias_ref, sk_ref, a2_ref, out_ref):
    la = (jnp.sum(ss_ref[...], axis=0)
          / jnp.maximum(jnp.sum(cnt_ref[...], axis=0), 1.0))
    h = _self_loop_combine(num_ref, den_ref, la, xl_ref, xr_ref, wev_ref,
                           att_ref, bias_ref)
    h = h + sk_ref[...]
    out_ref[...] = jnp.where(h >= 0, h, a2_ref[...] * h)

  row = pl.BlockSpec((_RB, _D), lambda i: (i, 0))
  vec = pl.BlockSpec((1, _D), lambda i: (0, 0))
  par = pl.BlockSpec((_NW, _RB), lambda i: (0, i))
  return pl.pallas_call(
      body,
      grid=(_NRB,),
      in_specs=[pl.BlockSpec((_NC, _RB, _D), lambda i: (0, i, 0)),
                par, par, par, row, row, vec, vec, vec, row, vec],
      out_specs=row,
      out_shape=jax.ShapeDtypeStruct((_NP, _D), jnp.float32),
  )(num2, den2, cnt, ssm, xl2, xr2, wev2, att2, bias2, skip, a2)


@jax.jit
def _impl(x, edge_index, edge_weight, W1l, b1l, W1r, b1r, att1, We1, bias1,
          W2l, b2l, W2r, b2r, att2, We2, bias2, Ws, bs, a1, a2):
  src = edge_index[0].reshape(_E // _BB, _BB)
  dst = edge_index[1].reshape(_E // _BB, _BB)
  ew = edge_weight[:, 0].reshape(_E // _BB, _BB)
  wev1 = We1[:, 0]
  attv1 = att1.reshape(-1)
  wev2 = We2[:, 0]
  attv2 = att2.reshape(-1)
  r1 = lambda v: v.reshape(1, _D)

  xp = jnp.pad(x, ((0, _NP - _N), (0, 0)))
  xl1, xr1, skip = _proj3(xp, W1l.T, r1(b1l), W1r.T, r1(b1r), Ws.T, r1(bs))
  num1, den1, cnt, ssm = _sc_pass_stats(xl1, xr1, src, dst, ew, wev1, attv1)
  xl2, xr2 = _combine1(num1, den1, cnt, ssm, xl1, xr1, r1(wev1), r1(attv1),
                       r1(bias1), r1(a1), W2l.T, r1(b2l), W2r.T, r1(b2r))
  num2, den2 = _sc_pass(xl2, xr2, src, dst, ew, wev2, attv2)
  out = _combine2(num2, den2, cnt, ssm, xl2, xr2, r1(wev2), r1(attv2),
                  r1(bias2), skip, r1(a2))
  return out[:_N]


def kernel(x, edge_index, edge_weight, W1l, b1l, W1r, b1r, att1, We1, bias1,
           W2l, b2l, W2r, b2r, att2, We2, bias2, Ws, bs, a1, a2):
  return _impl(x, edge_index, edge_weight, W1l, b1l, W1r, b1r, att1, We1,
               bias1, W2l, b2l, W2r, b2r, att2, We2, bias2, Ws, bs, a1, a2)
